# Initial kernel scaffold; baseline (speedup 1.0000x reference)
import jax
import jax.numpy as jnp
from jax.experimental import pallas as pl


def kernel(z, g_edge_index, h_edge_index, i_edge_index, W1, W2, W3, W4, W5, W6):
    raise NotImplementedError("write your pallas kernel here")


# speedup vs baseline: 11.4997x; 11.4997x over previous
import functools
import jax
import jax.numpy as jnp
from jax import lax
from jax.experimental import pallas as pl
from jax.experimental.pallas import tpu as pltpu
from jax.experimental.pallas import tpu_sc as plsc

N_G = 100000
E_G = 1600000
E_H = 1600000
E_I = 1000000
D = 64

NC, NS, L = 2, 16, 16
NW = NC * NS
CH = 128

_MESH = plsc.VectorSubcoreMesh(core_axis_name="c", subcore_axis_name="s",
                               num_cores=NC, num_subcores=NS)


def _wid():
    return lax.axis_index("s") * NC + lax.axis_index("c")


def _chunk_loop(n_edges, body_fn):
    n_chunks = -(-n_edges // CH)
    per_worker = -(-n_chunks // NW)
    wid = _wid()

    def body(j, _):
        gc = wid + j * NW

        @pl.when(gc < n_chunks)
        def _():
            base = jnp.minimum(gc * CH, n_edges - CH)
            body_fn(base)
        return 0

    lax.fori_loop(0, per_worker, body, 0)


def _gather2_body(n_edges, combine):
    def body(tab_hbm, ia_hbm, ib_hbm, o_hbm, ia_v, ib_v, a_v, b_v, o_v, sem):
        def chunk(base):
            pltpu.sync_copy(ia_hbm.at[pl.ds(base, CH)], ia_v)
            pltpu.sync_copy(ib_hbm.at[pl.ds(base, CH)], ib_v)
            pltpu.async_copy(tab_hbm.at[ia_v], a_v, sem).wait()
            pltpu.async_copy(tab_hbm.at[ib_v], b_v, sem).wait()
            for t in range(CH // L):
                s = pl.ds(t * L, L)
                o_v[s] = combine(a_v[s], b_v[s])
            pltpu.sync_copy(o_v, o_hbm.at[pl.ds(base, CH)])

        _chunk_loop(n_edges, chunk)

    return body


def _make_gather2(n_edges, combine):
    return functools.partial(
        pl.kernel,
        out_type=jax.ShapeDtypeStruct((n_edges,), jnp.int32),
        mesh=_MESH,
        scratch_types=[
            pltpu.VMEM((CH,), jnp.int32), pltpu.VMEM((CH,), jnp.int32),
            pltpu.VMEM((CH,), jnp.int32), pltpu.VMEM((CH,), jnp.int32),
            pltpu.VMEM((CH,), jnp.int32),
            pltpu.SemaphoreType.DMA,
        ],
    )(_gather2_body(n_edges, combine))


_k1 = _make_gather2(E_G, lambda a, b: (a << 1) | b)
_k2 = _make_gather2(E_H, lambda a, b: (a & 2) | (b & 1))
_k3 = _make_gather2(E_I, lambda a, b: (a << 2) | b)


_BT = 8192


def _expand_body(code_ref, w_ref, out_ref):
    codes = lax.broadcasted_iota(jnp.int32, (16, 1), 0)
    za, zc = (codes >> 3) & 1, (codes >> 2) & 1
    zb, zd = (codes >> 1) & 1, codes & 1
    w = w_ref[...]
    t16 = jnp.zeros((16, D), jnp.float32)
    for t, (x, y) in enumerate(((za, zc), (za, zb), (zc, zb),
                                (za, zd), (zc, zd), (zb, zd))):
        t16 = t16 + jnp.where(x == y, w[t, 1][None, :], w[t, 0][None, :])
    code = code_ref[...]
    onehot = (code[:, None] == lax.broadcasted_iota(jnp.int32, (1, 16), 1))
    out_ref[...] = jnp.dot(onehot.astype(jnp.float32), t16,
                           preferred_element_type=jnp.float32)


_expand = pl.pallas_call(
    _expand_body,
    grid=(-(-E_I // _BT),),
    in_specs=[
        pl.BlockSpec((_BT,), lambda i: (i,)),
        pl.BlockSpec((6, 2, D), lambda i: (0, 0, 0)),
    ],
    out_specs=pl.BlockSpec((_BT, D), lambda i: (i, 0)),
    out_shape=jax.ShapeDtypeStruct((E_I, D), jnp.float32),
)


def kernel(z, g_edge_index, h_edge_index, i_edge_index, W1, W2, W3, W4, W5, W6):
    z = z.astype(jnp.int32)
    g0 = g_edge_index[0].astype(jnp.int32)
    g1 = g_edge_index[1].astype(jnp.int32)
    h0 = h_edge_index[0].astype(jnp.int32)
    h1 = h_edge_index[1].astype(jnp.int32)
    i0 = i_edge_index[0].astype(jnp.int32)
    i1 = i_edge_index[1].astype(jnp.int32)
    w_stack = jnp.stack([W1, W2, W3, W4, W5, W6]).astype(jnp.float32)

    pair = _k1(z, g0, g1)
    c = _k2(pair, h0, h1)
    code = _k3(c, i0, i1)
    return _expand(code, w_stack)


# speedup vs baseline: 47.6822x; 4.1464x over previous
import functools
import jax
import jax.numpy as jnp
from jax import lax
from jax.experimental import pallas as pl
from jax.experimental.pallas import tpu as pltpu
from jax.experimental.pallas import tpu_sc as plsc

N_G = 100000
E_G = 1600000
E_H = 1600000
E_I = 1000000
D = 64

NC, NS, L = 2, 16, 16
NW = NC * NS
SCH = 2048

_MESH = plsc.VectorSubcoreMesh(core_axis_name="c", subcore_axis_name="s",
                               num_cores=NC, num_subcores=NS)


def _wid():
    return lax.axis_index("s") * NC + lax.axis_index("c")


def _ext2(tab_v, i):
    w = plsc.load_gather(tab_v, [i >> 4])
    return (w >> ((i & 15) << 1)) & 3


def _sc_body(n_edges, n_tab, pack_out, value_fn):
    n_chunks = -(-n_edges // SCH)
    shifts = None

    def body(tab_hbm, ia_hbm, ib_hbm, o_hbm, tab_v, ia_v, ib_v, o_v, sem):
        pltpu.sync_copy(tab_hbm, tab_v)
        li = lax.iota(jnp.int32, L)
        shifts = li << 1
        wid = _wid()

        def chunk(j, _):
            gc = wid + j * NW

            @pl.when(gc < n_chunks)
            def _():
                base = jnp.minimum(gc * SCH, n_edges - SCH)
                cp_a = pltpu.async_copy(ia_hbm.at[pl.ds(base, SCH)], ia_v, sem)
                cp_b = pltpu.async_copy(ib_hbm.at[pl.ds(base, SCH)], ib_v, sem)
                cp_a.wait()
                cp_b.wait()
                if pack_out:
                    for wslot in range(SCH // (L * L)):
                        wvec = jnp.zeros((L,), jnp.int32)
                        for l in range(L):
                            t = wslot * L + l
                            s = pl.ds(t * L, L)
                            v = value_fn(tab_v, ia_v[s], ib_v[s])
                            word = jnp.sum(v << shifts)
                            wvec = jnp.where(li == l, word, wvec)
                        o_v[pl.ds(wslot * L, L)] = wvec
                    wbase = pl.multiple_of(
                        jnp.minimum(gc * (SCH // L), (n_edges - SCH) // L), 8)
                    pltpu.sync_copy(o_v, o_hbm.at[pl.ds(wbase, SCH // L)])
                else:
                    for t in range(SCH // L):
                        s = pl.ds(t * L, L)
                        o_v[s] = value_fn(tab_v, ia_v[s], ib_v[s])
                    pltpu.sync_copy(o_v, o_hbm.at[pl.ds(base, SCH)])
            return 0

        lax.fori_loop(0, -(-n_chunks // NW), chunk, 0)

    n_out = n_edges // L if pack_out else n_edges
    return functools.partial(
        pl.kernel,
        out_type=jax.ShapeDtypeStruct((n_out,), jnp.int32),
        mesh=_MESH,
        scratch_types=[
            pltpu.VMEM((n_tab,), jnp.int32),
            pltpu.VMEM((SCH,), jnp.int32), pltpu.VMEM((SCH,), jnp.int32),
            pltpu.VMEM((SCH // L if pack_out else SCH,), jnp.int32),
            pltpu.SemaphoreType.DMA,
        ],
        compiler_params=pltpu.CompilerParams(needs_layout_passes=False),
    )(body)


_k1 = _sc_body(
    E_G, N_G, True,
    lambda tab_v, a, b: (plsc.load_gather(tab_v, [a]) << 1)
    | plsc.load_gather(tab_v, [b]))
_k2 = _sc_body(
    E_H, E_G // L, True,
    lambda tab_v, a, b: (_ext2(tab_v, a) & 2) | (_ext2(tab_v, b) & 1))
_k3 = _sc_body(
    E_I, E_H // L, False,
    lambda tab_v, a, b: (_ext2(tab_v, a) << 2) | _ext2(tab_v, b))


_BT = 8192


def _expand_body(code_ref, w_ref, out_ref):
    codes = lax.broadcasted_iota(jnp.int32, (16, 1), 0)
    za, zc = (codes >> 3) & 1, (codes >> 2) & 1
    zb, zd = (codes >> 1) & 1, codes & 1
    w = w_ref[...]
    t16 = jnp.zeros((16, D), jnp.float32)
    for t, (x, y) in enumerate(((za, zc), (za, zb), (zc, zb),
                                (za, zd), (zc, zd), (zb, zd))):
        t16 = t16 + jnp.where(x == y, w[t, 1][None, :], w[t, 0][None, :])
    code = code_ref[...]
    onehot = (code[:, None] == lax.broadcasted_iota(jnp.int32, (1, 16), 1))
    out_ref[...] = jnp.dot(onehot.astype(jnp.float32), t16,
                           preferred_element_type=jnp.float32)


_expand = pl.pallas_call(
    _expand_body,
    grid=(-(-E_I // _BT),),
    in_specs=[
        pl.BlockSpec((_BT,), lambda i: (i,)),
        pl.BlockSpec((6, 2, D), lambda i: (0, 0, 0)),
    ],
    out_specs=pl.BlockSpec((_BT, D), lambda i: (i, 0)),
    out_shape=jax.ShapeDtypeStruct((E_I, D), jnp.float32),
)


def kernel(z, g_edge_index, h_edge_index, i_edge_index, W1, W2, W3, W4, W5, W6):
    z = z.astype(jnp.int32)
    g0 = g_edge_index[0].astype(jnp.int32)
    g1 = g_edge_index[1].astype(jnp.int32)
    h0 = h_edge_index[0].astype(jnp.int32)
    h1 = h_edge_index[1].astype(jnp.int32)
    i0 = i_edge_index[0].astype(jnp.int32)
    i1 = i_edge_index[1].astype(jnp.int32)
    w_stack = jnp.stack([W1, W2, W3, W4, W5, W6]).astype(jnp.float32)

    pair_p = _k1(z, g0, g1)
    c_p = _k2(pair_p, h0, h1)
    code = _k3(c_p, i0, i1)
    return _expand(code, w_stack)


# speedup vs baseline: 49.0683x; 1.0291x over previous
import functools
import jax
import jax.numpy as jnp
from jax import lax
from jax.experimental import pallas as pl
from jax.experimental.pallas import tpu as pltpu
from jax.experimental.pallas import tpu_sc as plsc

N_G = 100000
E_G = 1600000
E_H = 1600000
E_I = 1000000
D = 64

NC, NS, L = 2, 16, 16
NW = NC * NS
SCH = 2048

_MESH = plsc.VectorSubcoreMesh(core_axis_name="c", subcore_axis_name="s",
                               num_cores=NC, num_subcores=NS)


def _wid():
    return lax.axis_index("s") * NC + lax.axis_index("c")


def _ext2(tab_v, i):
    w = plsc.load_gather(tab_v, [i >> 4])
    return (w >> ((i & 15) << 1)) & 3


def _sc_body(n_edges, n_tab, pack_out, value_fn):
    n_chunks = -(-n_edges // SCH)
    shifts = None

    def body(tab_hbm, ia_hbm, ib_hbm, o_hbm, tab_v, ia_v, ib_v, o_v, sem):
        pltpu.sync_copy(tab_hbm, tab_v)
        li = lax.iota(jnp.int32, L)
        shifts = li << 1
        wid = _wid()

        def chunk(j, _):
            gc = wid + j * NW

            @pl.when(gc < n_chunks)
            def _():
                base = jnp.minimum(gc * SCH, n_edges - SCH)
                cp_a = pltpu.async_copy(ia_hbm.at[pl.ds(base, SCH)], ia_v, sem)
                cp_b = pltpu.async_copy(ib_hbm.at[pl.ds(base, SCH)], ib_v, sem)
                cp_a.wait()
                cp_b.wait()
                if pack_out:
                    for wslot in range(SCH // (L * L)):
                        wvec = jnp.zeros((L,), jnp.int32)
                        for l in range(L):
                            t = wslot * L + l
                            s = pl.ds(t * L, L)
                            v = value_fn(tab_v, ia_v[s], ib_v[s])
                            word = jnp.sum(v << shifts)
                            wvec = jnp.where(li == l, word, wvec)
                        o_v[pl.ds(wslot * L, L)] = wvec
                    wbase = pl.multiple_of(
                        jnp.minimum(gc * (SCH // L), (n_edges - SCH) // L), 8)
                    pltpu.sync_copy(o_v, o_hbm.at[pl.ds(wbase, SCH // L)])
                else:
                    for t in range(SCH // L):
                        s = pl.ds(t * L, L)
                        o_v[s] = value_fn(tab_v, ia_v[s], ib_v[s])
                    pltpu.sync_copy(o_v, o_hbm.at[pl.ds(base, SCH)])
            return 0

        lax.fori_loop(0, -(-n_chunks // NW), chunk, 0)

    n_out = n_edges // L if pack_out else n_edges
    return functools.partial(
        pl.kernel,
        out_type=jax.ShapeDtypeStruct((n_out,), jnp.int32),
        mesh=_MESH,
        scratch_types=[
            pltpu.VMEM((n_tab,), jnp.int32),
            pltpu.VMEM((SCH,), jnp.int32), pltpu.VMEM((SCH,), jnp.int32),
            pltpu.VMEM((SCH // L if pack_out else SCH,), jnp.int32),
            pltpu.SemaphoreType.DMA,
        ],
        compiler_params=pltpu.CompilerParams(needs_layout_passes=False),
    )(body)


_k1 = _sc_body(
    E_G, N_G, True,
    lambda tab_v, a, b: (plsc.load_gather(tab_v, [a]) << 1)
    | plsc.load_gather(tab_v, [b]))
_k2 = _sc_body(
    E_H, E_G // L, True,
    lambda tab_v, a, b: (_ext2(tab_v, a) & 2) | (_ext2(tab_v, b) & 1))
_k3 = _sc_body(
    E_I, E_H // L, False,
    lambda tab_v, a, b: (_ext2(tab_v, a) << 2) | _ext2(tab_v, b))


_BT = 16384


def _expand_body(code_ref, w_ref, out_ref):
    codes = lax.broadcasted_iota(jnp.int32, (16, 1), 0)
    za, zc = (codes >> 3) & 1, (codes >> 2) & 1
    zb, zd = (codes >> 1) & 1, codes & 1
    w = w_ref[...]
    t16 = jnp.zeros((16, D), jnp.float32)
    for t, (x, y) in enumerate(((za, zc), (za, zb), (zc, zb),
                                (za, zd), (zc, zd), (zb, zd))):
        t16 = t16 + jnp.where(x == y, w[t, 1][None, :], w[t, 0][None, :])
    code = code_ref[...]
    onehot = (code[:, None] == lax.broadcasted_iota(jnp.int32, (1, 16), 1))
    out_ref[...] = jnp.dot(onehot.astype(jnp.float32), t16,
                           preferred_element_type=jnp.float32)


_expand = pl.pallas_call(
    _expand_body,
    grid=(-(-E_I // _BT),),
    in_specs=[
        pl.BlockSpec((_BT,), lambda i: (i,)),
        pl.BlockSpec((6, 2, D), lambda i: (0, 0, 0)),
    ],
    out_specs=pl.BlockSpec((_BT, D), lambda i: (i, 0)),
    out_shape=jax.ShapeDtypeStruct((E_I, D), jnp.float32),
)


def kernel(z, g_edge_index, h_edge_index, i_edge_index, W1, W2, W3, W4, W5, W6):
    z = z.astype(jnp.int32)
    g0 = g_edge_index[0].astype(jnp.int32)
    g1 = g_edge_index[1].astype(jnp.int32)
    h0 = h_edge_index[0].astype(jnp.int32)
    h1 = h_edge_index[1].astype(jnp.int32)
    i0 = i_edge_index[0].astype(jnp.int32)
    i1 = i_edge_index[1].astype(jnp.int32)
    w_stack = jnp.stack([W1, W2, W3, W4, W5, W6]).astype(jnp.float32)

    pair_p = _k1(z, g0, g1)
    c_p = _k2(pair_p, h0, h1)
    code = _k3(c_p, i0, i1)
    return _expand(code, w_stack)


# speedup vs baseline: 49.6464x; 1.0118x over previous
import functools
import jax
import jax.numpy as jnp
from jax import lax
from jax.experimental import pallas as pl
from jax.experimental.pallas import tpu as pltpu
from jax.experimental.pallas import tpu_sc as plsc

N_G = 100000
E_G = 1600000
E_H = 1600000
E_I = 1000000
D = 64

NC, NS, L = 2, 16, 16
NW = NC * NS
SCH = 2048

_MESH = plsc.VectorSubcoreMesh(core_axis_name="c", subcore_axis_name="s",
                               num_cores=NC, num_subcores=NS)


def _wid():
    return lax.axis_index("s") * NC + lax.axis_index("c")


def _ext2(tab_v, i):
    w = plsc.load_gather(tab_v, [i >> 4])
    return (w >> ((i & 15) << 1)) & 3


def _sc_body(n_edges, n_tab, pack_out, value_fn):
    n_chunks = -(-n_edges // SCH)
    shifts = None

    def body(tab_hbm, ia_hbm, ib_hbm, o_hbm, tab_v, ia_v, ib_v, o_v, sem):
        pltpu.sync_copy(tab_hbm, tab_v)
        li = lax.iota(jnp.int32, L)
        shifts = li << 1
        wid = _wid()

        def chunk(j, _):
            gc = wid + j * NW

            @pl.when(gc < n_chunks)
            def _():
                base = jnp.minimum(gc * SCH, n_edges - SCH)
                cp_a = pltpu.async_copy(ia_hbm.at[pl.ds(base, SCH)], ia_v, sem)
                cp_b = pltpu.async_copy(ib_hbm.at[pl.ds(base, SCH)], ib_v, sem)
                cp_a.wait()
                cp_b.wait()
                if pack_out:
                    for wslot in range(SCH // (L * L)):
                        wvec = jnp.zeros((L,), jnp.int32)
                        for l in range(L):
                            t = wslot * L + l
                            s = pl.ds(t * L, L)
                            v = value_fn(tab_v, ia_v[s], ib_v[s])
                            word = jnp.sum(v << shifts)
                            wvec = jnp.where(li == l, word, wvec)
                        o_v[pl.ds(wslot * L, L)] = wvec
                    wbase = pl.multiple_of(
                        jnp.minimum(gc * (SCH // L), (n_edges - SCH) // L), 8)
                    pltpu.sync_copy(o_v, o_hbm.at[pl.ds(wbase, SCH // L)])
                else:
                    for t in range(SCH // L):
                        s = pl.ds(t * L, L)
                        o_v[s] = value_fn(tab_v, ia_v[s], ib_v[s])
                    pltpu.sync_copy(o_v, o_hbm.at[pl.ds(base, SCH)])
            return 0

        lax.fori_loop(0, -(-n_chunks // NW), chunk, 0)

    n_out = n_edges // L if pack_out else n_edges
    return functools.partial(
        pl.kernel,
        out_type=jax.ShapeDtypeStruct((n_out,), jnp.int32),
        mesh=_MESH,
        scratch_types=[
            pltpu.VMEM((n_tab,), jnp.int32),
            pltpu.VMEM((SCH,), jnp.int32), pltpu.VMEM((SCH,), jnp.int32),
            pltpu.VMEM((SCH // L if pack_out else SCH,), jnp.int32),
            pltpu.SemaphoreType.DMA,
        ],
        compiler_params=pltpu.CompilerParams(needs_layout_passes=False),
    )(body)


_k1 = _sc_body(
    E_G, N_G, True,
    lambda tab_v, a, b: (plsc.load_gather(tab_v, [a]) << 1)
    | plsc.load_gather(tab_v, [b]))
_k2 = _sc_body(
    E_H, E_G // L, True,
    lambda tab_v, a, b: (_ext2(tab_v, a) & 2) | (_ext2(tab_v, b) & 1))
_k3 = _sc_body(
    E_I, E_H // L, False,
    lambda tab_v, a, b: (_ext2(tab_v, a) << 2) | _ext2(tab_v, b))


_BT = 25600


def _expand_body(code_ref, w_ref, out_ref):
    codes = lax.broadcasted_iota(jnp.int32, (16, 1), 0)
    za, zc = (codes >> 3) & 1, (codes >> 2) & 1
    zb, zd = (codes >> 1) & 1, codes & 1
    w = w_ref[...]
    t16 = jnp.zeros((16, D), jnp.float32)
    for t, (x, y) in enumerate(((za, zc), (za, zb), (zc, zb),
                                (za, zd), (zc, zd), (zb, zd))):
        t16 = t16 + jnp.where(x == y, w[t, 1][None, :], w[t, 0][None, :])
    code = code_ref[...]
    onehot = (code[:, None] == lax.broadcasted_iota(jnp.int32, (1, 16), 1))
    out_ref[...] = jnp.dot(onehot.astype(jnp.float32), t16,
                           preferred_element_type=jnp.float32)


_expand = pl.pallas_call(
    _expand_body,
    grid=(-(-E_I // _BT),),
    in_specs=[
        pl.BlockSpec((_BT,), lambda i: (i,)),
        pl.BlockSpec((6, 2, D), lambda i: (0, 0, 0)),
    ],
    out_specs=pl.BlockSpec((_BT, D), lambda i: (i, 0)),
    out_shape=jax.ShapeDtypeStruct((E_I, D), jnp.float32),
)


def kernel(z, g_edge_index, h_edge_index, i_edge_index, W1, W2, W3, W4, W5, W6):
    z = z.astype(jnp.int32)
    g0 = g_edge_index[0].astype(jnp.int32)
    g1 = g_edge_index[1].astype(jnp.int32)
    h0 = h_edge_index[0].astype(jnp.int32)
    h1 = h_edge_index[1].astype(jnp.int32)
    i0 = i_edge_index[0].astype(jnp.int32)
    i1 = i_edge_index[1].astype(jnp.int32)
    w_stack = jnp.stack([W1, W2, W3, W4, W5, W6]).astype(jnp.float32)

    pair_p = _k1(z, g0, g1)
    c_p = _k2(pair_p, h0, h1)
    code = _k3(c_p, i0, i1)
    return _expand(code, w_stack)
